# baseline (speedup 1.0000x reference)
import functools

import jax
import jax.numpy as jnp
from jax import lax
from jax.experimental import pallas as pl
from jax.experimental.pallas import tpu as pltpu
from jax.experimental.pallas import tpu_sc as plsc

_LANES = 16
_B = 128
_CPW = 50
_SPAN = _B * _CPW
_NW = 32


def _body(x_hbm, upx_hbm, idx_hbm, out_hbm,
          idx_all, g0, g1, u0, u1, o0, o1,
          sg0, sg1, su0, su1, so0, so1, *, m, n_col):
    gath = (g0, g1)
    upx = (u0, u1)
    outv = (o0, o1)
    sg = (sg0, sg1)
    su = (su0, su1)
    so = (so0, so1)

    wid = lax.axis_index("s") * 2 + lax.axis_index("c")
    pb = jnp.minimum(wid * _SPAN, m - _SPAN)
    pltpu.sync_copy(idx_hbm.at[pl.ds(pb, _SPAN)], idx_all)

    def in_copies(k, b):
        idx_slice = idx_all.at[pl.ds(k * _B, _B)]
        return (
            pltpu.make_async_copy(x_hbm.at[idx_slice], gath[b], sg[b]),
            pltpu.make_async_copy(
                upx_hbm.at[pl.ds(pb + k * _B, _B)], upx[b], su[b]),
        )

    def out_copy(k, b):
        return pltpu.make_async_copy(
            outv[b], out_hbm.at[pl.ds(pb + k * _B, _B)], so[b])

    for b in range(2):
        for cp in in_copies(b, b):
            cp.start()

    @pl.loop(0, _CPW, step=2)
    def _pair(k0):
        for b in range(2):
            k = k0 + b
            for cp in in_copies(k, b):
                cp.wait()

            @pl.when(k >= 2)
            def _():
                out_copy(k - 2, b).wait()

            @pl.loop(0, _B, unroll=2)
            def _row(r):
                for j in range(n_col // _LANES):
                    cc = j * _LANES
                    outv[b][r, pl.ds(cc, _LANES)] = (
                        upx[b][r, pl.ds(cc, _LANES)]
                        + gath[b][r, pl.ds(cc, _LANES)]
                    )

            out_copy(k, b).start()

            @pl.when(k + 2 < _CPW)
            def _():
                for cp in in_copies(k + 2, b):
                    cp.start()

    for b in range(2):
        out_copy(_CPW - 2 + b, b).wait()


def kernel(x, up_x, unq_inv):
    m, n_col = up_x.shape
    idx = unq_inv.astype(jnp.int32)
    assert _SPAN * _NW >= m and _SPAN <= m

    mesh = plsc.VectorSubcoreMesh(core_axis_name="c", subcore_axis_name="s")
    body = functools.partial(_body, m=m, n_col=n_col)
    run = pl.kernel(
        body,
        out_type=jax.ShapeDtypeStruct((m, n_col), jnp.float32),
        mesh=mesh,
        scratch_types=[
            pltpu.VMEM((_SPAN,), jnp.int32),
            pltpu.VMEM((_B, n_col), jnp.float32),
            pltpu.VMEM((_B, n_col), jnp.float32),
            pltpu.VMEM((_B, n_col), jnp.float32),
            pltpu.VMEM((_B, n_col), jnp.float32),
            pltpu.VMEM((_B, n_col), jnp.float32),
            pltpu.VMEM((_B, n_col), jnp.float32),
            pltpu.SemaphoreType.DMA,
            pltpu.SemaphoreType.DMA,
            pltpu.SemaphoreType.DMA,
            pltpu.SemaphoreType.DMA,
            pltpu.SemaphoreType.DMA,
            pltpu.SemaphoreType.DMA,
        ],
    )
    return run(x, up_x, idx)


# speedup vs baseline: 2.1858x; 2.1858x over previous
import functools

import jax
import jax.numpy as jnp
from jax import lax
from jax.experimental import pallas as pl
from jax.experimental.pallas import tpu as pltpu
from jax.experimental.pallas import tpu_sc as plsc

_LANES = 16
_B = 128
_CPW = 50
_SPAN = _B * _CPW
_NW = 32


def _body(x_hbm, upx_hbm, idx_hbm, out_hbm,
          idx_all, g0, g1, u0, u1, o0, o1,
          sg0, sg1, su0, su1, so0, so1, *, m, n_col):
    gath = (g0, g1)
    upx = (u0, u1)
    outv = (o0, o1)
    sg = (sg0, sg1)
    su = (su0, su1)
    so = (so0, so1)

    wid = lax.axis_index("s") * 2 + lax.axis_index("c")
    pb = jnp.minimum(wid * _SPAN, m - _SPAN)
    pltpu.sync_copy(idx_hbm.at[pl.ds(pb, _SPAN)], idx_all)

    def in_copies(k, b):
        idx_slice = idx_all.at[pl.ds(k * _B, _B)]
        return (
            pltpu.make_async_copy(x_hbm.at[idx_slice], gath[b], sg[b]),
            pltpu.make_async_copy(
                upx_hbm.at[pl.ds(pb + k * _B, _B)], upx[b], su[b]),
        )

    def out_copy(k, b):
        return pltpu.make_async_copy(
            outv[b], out_hbm.at[pl.ds(pb + k * _B, _B)], so[b])

    for b in range(2):
        for cp in in_copies(b, b):
            cp.start()

    @pl.loop(0, _CPW, step=2)
    def _pair(k0):
        for b in range(2):
            k = k0 + b
            for cp in in_copies(k, b):
                cp.wait()

            @pl.when(k >= 2)
            def _():
                out_copy(k - 2, b).wait()

            ngrp = n_col // _LANES

            @plsc.parallel_loop(0, _B, unroll=2)
            def _row(r):
                u = [upx[b][r, pl.ds(j * _LANES, _LANES)] for j in range(ngrp)]
                g = [gath[b][r, pl.ds(j * _LANES, _LANES)] for j in range(ngrp)]
                s = [u[j] + g[j] for j in range(ngrp)]
                for j in range(ngrp):
                    outv[b][r, pl.ds(j * _LANES, _LANES)] = s[j]

            out_copy(k, b).start()

            @pl.when(k + 2 < _CPW)
            def _():
                for cp in in_copies(k + 2, b):
                    cp.start()

    for b in range(2):
        out_copy(_CPW - 2 + b, b).wait()


def kernel(x, up_x, unq_inv):
    m, n_col = up_x.shape
    idx = unq_inv.astype(jnp.int32)
    assert _SPAN * _NW >= m and _SPAN <= m

    mesh = plsc.VectorSubcoreMesh(core_axis_name="c", subcore_axis_name="s")
    body = functools.partial(_body, m=m, n_col=n_col)
    run = pl.kernel(
        body,
        out_type=jax.ShapeDtypeStruct((m, n_col), jnp.float32),
        mesh=mesh,
        scratch_types=[
            pltpu.VMEM((_SPAN,), jnp.int32),
            pltpu.VMEM((_B, n_col), jnp.float32),
            pltpu.VMEM((_B, n_col), jnp.float32),
            pltpu.VMEM((_B, n_col), jnp.float32),
            pltpu.VMEM((_B, n_col), jnp.float32),
            pltpu.VMEM((_B, n_col), jnp.float32),
            pltpu.VMEM((_B, n_col), jnp.float32),
            pltpu.SemaphoreType.DMA,
            pltpu.SemaphoreType.DMA,
            pltpu.SemaphoreType.DMA,
            pltpu.SemaphoreType.DMA,
            pltpu.SemaphoreType.DMA,
            pltpu.SemaphoreType.DMA,
        ],
    )
    return run(x, up_x, idx)
